# baseline (speedup 1.0000x reference)
import functools

import jax
import jax.numpy as jnp
from jax import lax
from jax.experimental import pallas as pl
from jax.experimental.pallas import tpu as pltpu
from jax.experimental.pallas import tpu_sc as plsc

N_CORES = 2
N_SUBCORES = 16
NW = N_CORES * N_SUBCORES
LANES = 16
BATCH = 16384
D = 32
N_USERS = 1000000
N_BUSINESS = 100000
BPW = BATCH // NW
CHUNK = 128
NCHUNK = BPW // CHUNK
GRP = 8
SCORE_BLK = 65536


def _score_body(x_ref, w_ref, o_ref):
    o_ref[...] = jnp.sum(x_ref[...] * w_ref[...], axis=0)


@functools.lru_cache(maxsize=None)
def _score_call(n):
    return pl.pallas_call(
        _score_body,
        grid=(pl.cdiv(n, SCORE_BLK),),
        in_specs=[
            pl.BlockSpec((D, SCORE_BLK), lambda i: (0, i)),
            pl.BlockSpec((D, 1), lambda i: (0, 0)),
        ],
        out_specs=pl.BlockSpec((SCORE_BLK,), lambda i: (i,)),
        out_shape=jax.ShapeDtypeStruct((n,), jnp.float32),
    )


NBT = 782
NB_PAD = NBT * CHUNK


def _sc_bscore_body(bt_hbm, wsplat_hbm, sb_hbm, x_v, w_v, out_v, sem):
    wid = lax.axis_index("s") * N_CORES + lax.axis_index("c")
    pltpu.sync_copy(wsplat_hbm, w_v)
    start = wid * NBT // NW
    stop = (wid + 1) * NBT // NW

    def chunk(ch, _):
        pltpu.sync_copy(bt_hbm.at[:, pl.ds(ch * CHUNK, CHUNK)], x_v)
        for o in range(0, CHUNK, LANES):
            acc = jnp.zeros((LANES,), jnp.float32)
            for c in range(D):
                acc = acc + x_v[c, pl.ds(o, LANES)] * w_v[c, pl.ds(0, LANES)]
            out_v[pl.ds(o, LANES)] = acc
        pltpu.sync_copy(out_v, sb_hbm.at[pl.ds(ch * CHUNK, CHUNK)])
        return _

    lax.fori_loop(start, stop, chunk, 0)


@functools.lru_cache(maxsize=None)
def _sc_bscore_call():
    return pl.kernel(
        _sc_bscore_body,
        out_type=jax.ShapeDtypeStruct((NB_PAD,), jnp.float32),
        mesh=plsc.VectorSubcoreMesh(core_axis_name="c", subcore_axis_name="s",
                                    num_cores=N_CORES,
                                    num_subcores=N_SUBCORES),
        compiler_params=pltpu.CompilerParams(needs_layout_passes=False,
                                             use_tc_tiling_on_sc=True),
        scratch_types=[
            pltpu.VMEM((D, CHUNK), jnp.float32),
            pltpu.VMEM((D, CHUNK), jnp.float32),
            pltpu.VMEM((CHUNK,), jnp.float32),
            pltpu.SemaphoreType.DMA,
        ],
    )


def _sc_gather_body_full(users_hbm, business_hbm, su_hbm, sb_hbm, bias_hbm,
                         out_hbm, uidx_v, bidx_v, uhi_v, bhi_v, su_v, sb_v,
                         bias_v, out_v, sem):
    wid = lax.axis_index("s") * N_CORES + lax.axis_index("c")
    base = wid * NCHUNK

    pltpu.sync_copy(users_hbm.at[pl.ds(base, NCHUNK)], uidx_v)
    pltpu.sync_copy(business_hbm.at[pl.ds(base, NCHUNK)], bidx_v)
    pltpu.sync_copy(bias_hbm, bias_v)

    for j in range(NCHUNK):
        for o in range(0, CHUNK, LANES):
            uhi_v[j, pl.ds(o, LANES)] = uidx_v[j, pl.ds(o, LANES)] >> 3
            bhi_v[j, pl.ds(o, LANES)] = bidx_v[j, pl.ds(o, LANES)] >> 3

    handles = []
    for j in range(NCHUNK):
        handles.append(pltpu.async_copy(
            su_hbm.at[uhi_v.at[j]], su_v.at[pl.ds(j * CHUNK, CHUNK)], sem))
        handles.append(pltpu.async_copy(
            sb_hbm.at[bhi_v.at[j]], sb_v.at[pl.ds(j * CHUNK, CHUNK)], sem))
    for h in handles:
        h.wait()

    bias_vec = bias_v[...]
    lane = lax.iota(jnp.int32, LANES)
    seven = jnp.full((LANES,), 7, jnp.int32)

    def group(g, _):
        j = g // (CHUNK // LANES)
        o = (g % (CHUNK // LANES)) * LANES
        i_vec = g * LANES + lane
        ulo = uidx_v[j, pl.ds(o, LANES)] & seven
        blo = bidx_v[j, pl.ds(o, LANES)] & seven
        us = plsc.load_gather(su_v, [i_vec, ulo])
        bs = plsc.load_gather(sb_v, [i_vec, blo])
        out_v[pl.ds(g * LANES, LANES)] = us + bs + bias_vec
        return _

    lax.fori_loop(0, BPW // LANES, group, 0)

    pltpu.sync_copy(out_v, out_hbm.at[pl.ds(wid * BPW, BPW)])


@functools.lru_cache(maxsize=None)
def _sc_gather_call():
    return pl.kernel(
        _sc_gather_body_full,
        out_type=jax.ShapeDtypeStruct((BATCH,), jnp.float32),
        mesh=plsc.VectorSubcoreMesh(core_axis_name="c", subcore_axis_name="s",
                                    num_cores=N_CORES,
                                    num_subcores=N_SUBCORES),
        compiler_params=pltpu.CompilerParams(needs_layout_passes=False,
                                             use_tc_tiling_on_sc=False),
        scratch_types=[
            pltpu.VMEM((NCHUNK, CHUNK), jnp.int32),
            pltpu.VMEM((NCHUNK, CHUNK), jnp.int32),
            pltpu.VMEM((NCHUNK, CHUNK), jnp.int32),
            pltpu.VMEM((NCHUNK, CHUNK), jnp.int32),
            pltpu.VMEM((BPW, GRP), jnp.float32),
            pltpu.VMEM((BPW, GRP), jnp.float32),
            pltpu.VMEM((LANES,), jnp.float32),
            pltpu.VMEM((BPW,), jnp.float32),
            pltpu.SemaphoreType.DMA,
        ],
    )


def kernel(users, business, user_table, business_table, W, b):
    users2 = users.astype(jnp.int32).reshape(NW * NCHUNK, CHUNK)
    business2 = business.astype(jnp.int32).reshape(NW * NCHUNK, CHUNK)
    wu = W[0, :D].reshape(D, 1)
    wb = W[0, D:].reshape(D, 1)
    wsplat_b = jnp.broadcast_to(wb, (D, CHUNK))
    sb = _sc_bscore_call()(business_table.T, wsplat_b)
    su = _score_call(N_USERS)(user_table.T, wu)
    su2 = su.reshape(N_USERS // GRP, GRP)
    sb2 = sb.reshape(NB_PAD // GRP, GRP)
    b16 = jnp.broadcast_to(b.reshape(()), (LANES,))
    out = _sc_gather_call()(users2, business2, su2, sb2, b16)
    return out.reshape(BATCH, 1)


# speedup vs baseline: 1.1136x; 1.1136x over previous
import functools

import jax
import jax.numpy as jnp
from jax import lax
from jax.experimental import pallas as pl
from jax.experimental.pallas import tpu as pltpu
from jax.experimental.pallas import tpu_sc as plsc

N_CORES = 2
N_SUBCORES = 16
NW = N_CORES * N_SUBCORES
LANES = 16
BATCH = 16384
D = 32
N_USERS = 1000000
N_BUSINESS = 100000
BPW = BATCH // NW
CHUNK = 128
NCHUNK = BPW // CHUNK
GRP = 8
SCORE_BLK = 65536


def _score_body(x_ref, w_ref, o_ref):
    o_ref[...] = jnp.sum(x_ref[...] * w_ref[...], axis=0)


@functools.lru_cache(maxsize=None)
def _score_call(n):
    return pl.pallas_call(
        _score_body,
        grid=(pl.cdiv(n, SCORE_BLK),),
        in_specs=[
            pl.BlockSpec((D, SCORE_BLK), lambda i: (0, i)),
            pl.BlockSpec((D, 1), lambda i: (0, 0)),
        ],
        out_specs=pl.BlockSpec((SCORE_BLK,), lambda i: (i,)),
        out_shape=jax.ShapeDtypeStruct((n,), jnp.float32),
    )


def _sc_gather_body_full(users_hbm, business_hbm, su_hbm, sb_hbm, bias_hbm,
                         out_hbm, uidx_v, bidx_v, uhi_v, bhi_v, su_v, sb_v,
                         bias_v, out_v, sem):
    wid = lax.axis_index("s") * N_CORES + lax.axis_index("c")
    base = wid * NCHUNK

    pltpu.sync_copy(users_hbm.at[pl.ds(base, NCHUNK)], uidx_v)
    pltpu.sync_copy(business_hbm.at[pl.ds(base, NCHUNK)], bidx_v)
    pltpu.sync_copy(bias_hbm, bias_v)

    for j in range(NCHUNK):
        for o in range(0, CHUNK, LANES):
            uhi_v[j, pl.ds(o, LANES)] = uidx_v[j, pl.ds(o, LANES)] >> 3
            bhi_v[j, pl.ds(o, LANES)] = bidx_v[j, pl.ds(o, LANES)] >> 3

    handles = []
    for j in range(NCHUNK):
        handles.append(pltpu.async_copy(
            su_hbm.at[uhi_v.at[j]], su_v.at[pl.ds(j * CHUNK, CHUNK)], sem))
        handles.append(pltpu.async_copy(
            sb_hbm.at[bhi_v.at[j]], sb_v.at[pl.ds(j * CHUNK, CHUNK)], sem))
    for h in handles:
        h.wait()

    bias_vec = bias_v[...]
    lane = lax.iota(jnp.int32, LANES)
    seven = jnp.full((LANES,), 7, jnp.int32)

    def group(g, _):
        j = g // (CHUNK // LANES)
        o = (g % (CHUNK // LANES)) * LANES
        i_vec = g * LANES + lane
        ulo = uidx_v[j, pl.ds(o, LANES)] & seven
        blo = bidx_v[j, pl.ds(o, LANES)] & seven
        us = plsc.load_gather(su_v, [i_vec, ulo])
        bs = plsc.load_gather(sb_v, [i_vec, blo])
        out_v[pl.ds(g * LANES, LANES)] = us + bs + bias_vec
        return _

    lax.fori_loop(0, BPW // LANES, group, 0)

    pltpu.sync_copy(out_v, out_hbm.at[pl.ds(wid * BPW, BPW)])


@functools.lru_cache(maxsize=None)
def _sc_gather_call():
    return pl.kernel(
        _sc_gather_body_full,
        out_type=jax.ShapeDtypeStruct((BATCH,), jnp.float32),
        mesh=plsc.VectorSubcoreMesh(core_axis_name="c", subcore_axis_name="s",
                                    num_cores=N_CORES,
                                    num_subcores=N_SUBCORES),
        compiler_params=pltpu.CompilerParams(needs_layout_passes=False,
                                             use_tc_tiling_on_sc=False),
        scratch_types=[
            pltpu.VMEM((NCHUNK, CHUNK), jnp.int32),
            pltpu.VMEM((NCHUNK, CHUNK), jnp.int32),
            pltpu.VMEM((NCHUNK, CHUNK), jnp.int32),
            pltpu.VMEM((NCHUNK, CHUNK), jnp.int32),
            pltpu.VMEM((BPW, GRP), jnp.float32),
            pltpu.VMEM((BPW, GRP), jnp.float32),
            pltpu.VMEM((LANES,), jnp.float32),
            pltpu.VMEM((BPW,), jnp.float32),
            pltpu.SemaphoreType.DMA,
        ],
    )


def kernel(users, business, user_table, business_table, W, b):
    users2 = users.astype(jnp.int32).reshape(NW * NCHUNK, CHUNK)
    business2 = business.astype(jnp.int32).reshape(NW * NCHUNK, CHUNK)
    wu = W[0, :D].reshape(D, 1)
    wb = W[0, D:].reshape(D, 1)
    su = _score_call(N_USERS)(user_table.T, wu)
    sb = _score_call(N_BUSINESS)(business_table.T, wb)
    su2 = su.reshape(N_USERS // GRP, GRP)
    sb2 = sb.reshape(N_BUSINESS // GRP, GRP)
    b16 = jnp.broadcast_to(b.reshape(()), (LANES,))
    out = _sc_gather_call()(users2, business2, su2, sb2, b16)
    return out.reshape(BATCH, 1)
